# baseline (speedup 1.0000x reference)
import functools

import jax
import jax.numpy as jnp
from jax import lax
from jax.experimental import pallas as pl
from jax.experimental.pallas import tpu as pltpu
from jax.experimental.pallas import tpu_sc as plsc

F = 26
V = 100000
D = 16
B = 16384

NC = 2
NS = 16
NW = NC * NS

P = F * D
PPW = P // NW
L = 16

V0 = 49152
V1 = V - V0 - 32
TAIL = 32
QB = B // 4
U = 4


def _esmm_kernel(batch_t, tab_t, tail_t, out_t, plane_v, idx_v, out_a, out_b,
                 out_c, tail_v, sem0, sem1, semo):
    wid = lax.axis_index("s") * NC + lax.axis_index("c")

    def fd(j):
        p = wid * PPW + j
        return p, p // D, p % D

    def fire_h0(j):
        _, f, d = fd(j)
        return pltpu.async_copy(tab_t.at[f, d, pl.ds(0, V0)],
                                plane_v.at[pl.ds(0, V0)], sem0)

    def fire_h1(j):
        _, f, d = fd(j)
        return pltpu.async_copy(tab_t.at[f, d, pl.ds(V0, V1)],
                                plane_v.at[pl.ds(V0, V1)], sem1)

    def load_field(f):
        pltpu.sync_copy(tail_t.at[f], tail_v)
        pltpu.sync_copy(batch_t.at[f], idx_v)

    def pass_lo(q, buf):
        def body(i, c):
            for u in range(U):
                o = i * U * L + u * L
                iv = idx_v[pl.ds(q * QB + o, L)]
                m = iv < V0
                vals = plsc.load_gather(plane_v, [iv], mask=m)
                buf[pl.ds(o, L)] = vals
            return c

        lax.fori_loop(0, QB // (U * L), body, 0)

    def pass_hi(q, buf):
        def body(i, c):
            for u in range(U):
                o = i * U * L + u * L
                iv = idx_v[pl.ds(q * QB + o, L)]
                m = iv >= V0
                vals = plsc.load_gather(plane_v, [iv], mask=m)
                buf[pl.ds(o, L)] = jnp.where(m, vals, buf[pl.ds(o, L)])
            return c

        lax.fori_loop(0, QB // (U * L), body, 0)

    def fire_out(p, q, buf):
        return pltpu.async_copy(buf, out_t.at[p, pl.ds(q * QB, QB)], semo)

    cp_h0 = fire_h0(0)
    cp_h1 = fire_h1(0)
    oq = [None, None, None, None]
    for j in range(PPW):
        p, f, d = fd(j)
        if j == 0:
            load_field(f)
        else:
            _, f_prev, _ = fd(j - 1)

            @pl.when(f != f_prev)
            def _():
                load_field(f)

        plane_v[pl.ds(V - 2 * L, L)] = tail_v[pl.ds(d * TAIL, L)]
        plane_v[pl.ds(V - L, L)] = tail_v[pl.ds(d * TAIL + L, L)]
        if oq[1] is not None:
            oq[1].wait()
        if oq[2] is not None:
            oq[2].wait()
        cp_h0.wait()
        if oq[3] is not None:
            oq[3].wait()
        pass_lo(0, out_a)
        pass_lo(1, out_b)
        pass_lo(2, out_c)
        cp_h1.wait()
        pass_hi(0, out_a)
        oq[0] = fire_out(p, 0, out_a)
        oq[0].wait()
        pass_lo(3, out_a)
        if j + 1 < PPW:
            cp_h0 = fire_h0(j + 1)
        pass_hi(1, out_b)
        oq[1] = fire_out(p, 1, out_b)
        pass_hi(2, out_c)
        oq[2] = fire_out(p, 2, out_c)
        pass_hi(3, out_a)
        oq[3] = fire_out(p, 3, out_a)
        if j + 1 < PPW:
            cp_h1 = fire_h1(j + 1)
    oq[1].wait()
    oq[2].wait()
    oq[3].wait()


@jax.jit
def _esmm(batch, tables):
    batch_t = batch.astype(jnp.int32).T
    tab_t = tables.transpose(0, 2, 1)
    tail_t = lax.slice(tables, (0, V - TAIL, 0), (F, V, D)).transpose(
        0, 2, 1).reshape(F, D * TAIL)
    mesh = plsc.VectorSubcoreMesh(core_axis_name="c", subcore_axis_name="s")
    out_t = pl.kernel(
        _esmm_kernel,
        out_type=jax.ShapeDtypeStruct((P, B), jnp.float32),
        mesh=mesh,
        scratch_types=[
            pltpu.VMEM((V,), jnp.float32),
            pltpu.VMEM((B,), jnp.int32),
            pltpu.VMEM((QB,), jnp.float32),
            pltpu.VMEM((QB,), jnp.float32),
            pltpu.VMEM((QB,), jnp.float32),
            pltpu.VMEM((D * TAIL,), jnp.float32),
            pltpu.SemaphoreType.DMA,
            pltpu.SemaphoreType.DMA,
            pltpu.SemaphoreType.DMA,
        ],
        compiler_params=pltpu.CompilerParams(
            use_tc_tiling_on_sc=True, needs_layout_passes=False),
    )(batch_t, tab_t, tail_t)
    return out_t.T.reshape(B, F * D)


def kernel(batch, tables):
    return _esmm(batch, tables)


# speedup vs baseline: 1.6946x; 1.6946x over previous
import functools

import jax
import jax.numpy as jnp
from jax import lax
from jax.experimental import pallas as pl
from jax.experimental.pallas import tpu as pltpu
from jax.experimental.pallas import tpu_sc as plsc

F = 26
V = 100000
D = 16
B = 16384

NC = 2
NS = 16
NW = NC * NS

P = F * D
PPW = P // NW
L = 16

VA = 99968
TAIL = 32
HB = B // 2
U = 8


def _esmm_kernel(batch_t, tab_t, tail_t, out_t, plane_v, idx_v, out_v, tail_v,
                 semp, semo):
    wid = lax.axis_index("s") * NC + lax.axis_index("c")

    def fd(j):
        p = wid * PPW + j
        return p, p // D, p % D

    def fire_plane(j):
        _, f, d = fd(j)
        return pltpu.async_copy(tab_t.at[f, d, pl.ds(0, VA)],
                                plane_v.at[pl.ds(0, VA)], semp)

    def gather_half(h):
        def body(i, c):
            for u in range(U):
                o = i * U * L + u * L
                iv = idx_v[pl.ds(o, L)]
                out_v[pl.ds(h * HB + o, L)] = plsc.load_gather(plane_v, [iv])
            return c

        lax.fori_loop(0, HB // (U * L), body, 0)

    cp = fire_plane(0)
    ocp = None
    for j in range(PPW):
        p, f, d = fd(j)
        pltpu.sync_copy(tail_t.at[f], tail_v)
        plane_v[pl.ds(V - 2 * L, L)] = tail_v[pl.ds(d * TAIL, L)]
        plane_v[pl.ds(V - L, L)] = tail_v[pl.ds(d * TAIL + L, L)]
        pltpu.sync_copy(batch_t.at[f, pl.ds(0, HB)], idx_v)
        cp.wait()
        if ocp is not None:
            ocp.wait()
        gather_half(0)
        pltpu.sync_copy(batch_t.at[f, pl.ds(HB, HB)], idx_v)
        gather_half(1)
        ocp = pltpu.async_copy(out_v, out_t.at[p], semo)
        if j + 1 < PPW:
            cp = fire_plane(j + 1)
    ocp.wait()


@jax.jit
def _esmm(batch, tables):
    batch_t = batch.astype(jnp.int32).T
    tab_t = tables.transpose(0, 2, 1)
    tail_t = lax.slice(tables, (0, V - TAIL, 0), (F, V, D)).transpose(
        0, 2, 1).reshape(F, D * TAIL)
    mesh = plsc.VectorSubcoreMesh(core_axis_name="c", subcore_axis_name="s")
    out_t = pl.kernel(
        _esmm_kernel,
        out_type=jax.ShapeDtypeStruct((P, B), jnp.float32),
        mesh=mesh,
        scratch_types=[
            pltpu.VMEM((V,), jnp.float32),
            pltpu.VMEM((HB,), jnp.int32),
            pltpu.VMEM((B,), jnp.float32),
            pltpu.VMEM((D * TAIL,), jnp.float32),
            pltpu.SemaphoreType.DMA,
            pltpu.SemaphoreType.DMA,
        ],
        compiler_params=pltpu.CompilerParams(
            use_tc_tiling_on_sc=True, needs_layout_passes=False),
    )(batch_t, tab_t, tail_t)
    return out_t.T.reshape(B, F * D)


def kernel(batch, tables):
    return _esmm(batch, tables)


# speedup vs baseline: 2.2106x; 1.3045x over previous
import functools

import jax
import jax.numpy as jnp
from jax import lax
from jax.experimental import pallas as pl
from jax.experimental.pallas import tpu as pltpu
from jax.experimental.pallas import tpu_sc as plsc

F = 26
V = 100000
D = 16
B = 16384

NC = 2
NS = 16
NW = NC * NS

P = F * D
PPW = P // NW
L = 16

VA = 99968
TAIL = 32
HB = B // 2
U = 8


def _esmm_kernel(batch_t, tab_t, tail_t, out_t, plane_v, idx_v, out_v, tail_v,
                 semp, semo):
    wid = lax.axis_index("s") * NC + lax.axis_index("c")

    def fd(j):
        p = wid * PPW + j
        return p, p // D, p % D

    def fire_plane(j):
        _, f, d = fd(j)
        return pltpu.async_copy(tab_t.at[f, d, pl.ds(0, VA)],
                                plane_v.at[pl.ds(0, VA)], semp)

    def gather_half(h):
        @plsc.parallel_loop(0, HB, step=L, unroll=U)
        def _(o):
            iv = idx_v[pl.ds(o, L)]
            out_v[pl.ds(h * HB + o, L)] = plsc.load_gather(plane_v, [iv])

    cp = fire_plane(0)
    ocp = None
    for j in range(PPW):
        p, f, d = fd(j)
        pltpu.sync_copy(tail_t.at[f], tail_v)
        plane_v[pl.ds(V - 2 * L, L)] = tail_v[pl.ds(d * TAIL, L)]
        plane_v[pl.ds(V - L, L)] = tail_v[pl.ds(d * TAIL + L, L)]
        pltpu.sync_copy(batch_t.at[f, pl.ds(0, HB)], idx_v)
        cp.wait()
        if ocp is not None:
            ocp.wait()
        gather_half(0)
        pltpu.sync_copy(batch_t.at[f, pl.ds(HB, HB)], idx_v)
        gather_half(1)
        ocp = pltpu.async_copy(out_v, out_t.at[p], semo)
        if j + 1 < PPW:
            cp = fire_plane(j + 1)
    ocp.wait()


@jax.jit
def _esmm(batch, tables):
    batch_t = batch.astype(jnp.int32).T
    tab_t = tables.transpose(0, 2, 1)
    tail_t = lax.slice(tables, (0, V - TAIL, 0), (F, V, D)).transpose(
        0, 2, 1).reshape(F, D * TAIL)
    mesh = plsc.VectorSubcoreMesh(core_axis_name="c", subcore_axis_name="s")
    out_t = pl.kernel(
        _esmm_kernel,
        out_type=jax.ShapeDtypeStruct((P, B), jnp.float32),
        mesh=mesh,
        scratch_types=[
            pltpu.VMEM((V,), jnp.float32),
            pltpu.VMEM((HB,), jnp.int32),
            pltpu.VMEM((B,), jnp.float32),
            pltpu.VMEM((D * TAIL,), jnp.float32),
            pltpu.SemaphoreType.DMA,
            pltpu.SemaphoreType.DMA,
        ],
        compiler_params=pltpu.CompilerParams(
            use_tc_tiling_on_sc=True, needs_layout_passes=False),
    )(batch_t, tab_t, tail_t)
    return out_t.T.reshape(B, F * D)


def kernel(batch, tables):
    return _esmm(batch, tables)
